# baseline (speedup 1.0000x reference)
import jax
import jax.numpy as jnp
from jax import lax
from jax.experimental import pallas as pl
from jax.experimental.pallas import tpu as pltpu
from jax.experimental.pallas import tpu_sc as plsc

DIM = 768
NUM_EXPERTS = 64
HIDDEN = 2 * DIM
L = 16


def _logits_kernel(x_ref, wr_ref, lt_ref):
    lt_ref[...] = jax.lax.dot_general(
        wr_ref[...], x_ref[...], (((1,), (1,)), ((), ())),
        preferred_element_type=jnp.float32)


def _sc_router(lt_hbm, idx_out, wts_out, eid_out, na_out,
               lt_v, idx_v, wts_v, eid_v, misc_v):
    cid = lax.axis_index("c")
    sid = lax.axis_index("s")

    @pl.when((cid == 0) & (sid == 0))
    def _():
        pltpu.sync_copy(lt_hbm, lt_v)
        avecs = []
        for g in range(4):
            m1 = jnp.full((L,), -jnp.inf, jnp.float32)
            m2 = jnp.full((L,), -jnp.inf, jnp.float32)
            a1 = jnp.zeros((L,), jnp.int32)
            a2 = jnp.zeros((L,), jnp.int32)
            for e in range(NUM_EXPERTS):
                v = lt_v[pl.ds(e * 64 + g * L, L)]
                gt1 = v > m1
                gt2 = v > m2
                ev = jnp.full((L,), e, jnp.int32)
                m2n = jnp.where(gt1, m1, jnp.where(gt2, v, m2))
                a2n = jnp.where(gt1, a1, jnp.where(gt2, ev, a2))
                m1 = jnp.where(gt1, v, m1)
                a1 = jnp.where(gt1, ev, a1)
                m2, a2 = m2n, a2n
            ed = jnp.exp(m2 - m1)
            r = 1.0 + ed
            wts_v[pl.ds(g * L, L)] = 1.0 / r
            wts_v[pl.ds(64 + g * L, L)] = ed / r
            idx_v[pl.ds(g * L, L)] = a1
            idx_v[pl.ds(64 + g * L, L)] = a2
            avecs.append(a1)
            avecs.append(a2)
        zi = jnp.zeros((L,), jnp.int32)
        ids = [lax.iota(jnp.int32, L) + c * L for c in range(4)]
        actv = [jnp.zeros((L,), jnp.int32) for _ in range(4)]
        for av in avecs:
            for l in range(L):
                et = av[l]
                etv = zi + et
                for c in range(4):
                    actv[c] = jnp.maximum(
                        actv[c], jnp.where(ids[c] == etv, 1, 0))
        na_v = jnp.zeros((L,), jnp.int32)
        lastv = jnp.zeros((L,), jnp.int32)
        steps = ids
        eidv = [jnp.zeros((L,), jnp.int32) for _ in range(4)]
        for e in range(NUM_EXPERTS):
            asv = zi + actv[e // L][e % L]
            ev = jnp.full((L,), e, jnp.int32)
            for c in range(4):
                cond = jnp.where(steps[c] == na_v, asv, zi)
                eidv[c] = jnp.where(cond > 0, ev, eidv[c])
            lastv = jnp.where(asv > 0, ev, lastv)
            na_v = na_v + asv
        for c in range(4):
            eid_v[pl.ds(c * L, L)] = jnp.where(steps[c] >= na_v, lastv,
                                               eidv[c])
        misc_v[...] = na_v
        pltpu.sync_copy(idx_v, idx_out)
        pltpu.sync_copy(wts_v, wts_out)
        pltpu.sync_copy(eid_v.at[pl.ds(0, NUM_EXPERTS)], eid_out)
        pltpu.sync_copy(misc_v.at[pl.ds(0, 8)], na_out)


def _moe_kernel(eid_ref, na_ref, x_ref, idx_ref, wts_ref,
                wg_ref, wu_ref, wd_ref, out_ref):
    i = pl.program_id(0)
    na = na_ref[0]

    @pl.when(i < na)
    def _():
        e = eid_ref[i]
        x = x_ref[...]
        gate_h = jax.lax.dot_general(
            x, wg_ref[0], (((1,), (1,)), ((), ())),
            preferred_element_type=jnp.float32)
        up_h = jax.lax.dot_general(
            x, wu_ref[0], (((1,), (1,)), ((), ())),
            preferred_element_type=jnp.float32)
        h = (gate_h * jax.nn.sigmoid(gate_h)) * up_h
        o = jax.lax.dot_general(
            h, wd_ref[0], (((1,), (1,)), ((), ())),
            preferred_element_type=jnp.float32)
        g2 = jnp.where(idx_ref[...] == e, wts_ref[...], 0.0)
        gate = g2[0, :] + g2[1, :]
        contrib = o * gate[:, None]

        @pl.when(i == 0)
        def _():
            out_ref[...] = contrib

        @pl.when(i > 0)
        def _():
            out_ref[...] += contrib


@jax.jit
def kernel(x, W_router, W_gate, W_up, W_down):
    orig_shape = x.shape
    x2 = x.reshape(-1, DIM)
    n = x2.shape[0]

    lt = pl.pallas_call(
        _logits_kernel,
        out_shape=jax.ShapeDtypeStruct((NUM_EXPERTS, n), jnp.float32),
    )(x2, W_router).reshape(NUM_EXPERTS * n)

    sc_router = pl.kernel(
        _sc_router,
        out_type=(
            jax.ShapeDtypeStruct((2 * n,), jnp.int32),
            jax.ShapeDtypeStruct((2 * n,), jnp.float32),
            jax.ShapeDtypeStruct((NUM_EXPERTS,), jnp.int32),
            jax.ShapeDtypeStruct((8,), jnp.int32),
        ),
        mesh=plsc.VectorSubcoreMesh(core_axis_name="c", subcore_axis_name="s"),
        scratch_types=(
            pltpu.VMEM((NUM_EXPERTS * 64,), jnp.float32),
            pltpu.VMEM((2 * 64,), jnp.int32),
            pltpu.VMEM((2 * 64,), jnp.float32),
            pltpu.VMEM((NUM_EXPERTS + 8,), jnp.int32),
            pltpu.VMEM((L,), jnp.int32),
        ),
    )
    idx_f, wts_f, eid, na = sc_router(lt)
    idx = idx_f.reshape(2, n)
    wts = wts_f.reshape(2, n)

    grid_spec = pltpu.PrefetchScalarGridSpec(
        num_scalar_prefetch=2,
        grid=(NUM_EXPERTS,),
        in_specs=[
            pl.BlockSpec((n, DIM), lambda i, eid, na: (0, 0)),
            pl.BlockSpec((2, n), lambda i, eid, na: (0, 0)),
            pl.BlockSpec((2, n), lambda i, eid, na: (0, 0)),
            pl.BlockSpec((1, HIDDEN, DIM), lambda i, eid, na: (eid[i], 0, 0)),
            pl.BlockSpec((1, HIDDEN, DIM), lambda i, eid, na: (eid[i], 0, 0)),
            pl.BlockSpec((1, DIM, HIDDEN), lambda i, eid, na: (eid[i], 0, 0)),
        ],
        out_specs=pl.BlockSpec((n, DIM), lambda i, eid, na: (0, 0)),
    )
    out = pl.pallas_call(
        _moe_kernel,
        grid_spec=grid_spec,
        out_shape=jax.ShapeDtypeStruct((n, DIM), jnp.float32),
        compiler_params=pltpu.CompilerParams(
            dimension_semantics=("arbitrary",),
        ),
    )(eid, na, x2, idx, wts, W_gate, W_up, W_down)

    return out.reshape(orig_shape)


# speedup vs baseline: 1.0064x; 1.0064x over previous
import jax
import jax.numpy as jnp
from jax import lax
from jax.experimental import pallas as pl
from jax.experimental.pallas import tpu as pltpu
from jax.experimental.pallas import tpu_sc as plsc

DIM = 768
NUM_EXPERTS = 64
HIDDEN = 2 * DIM
L = 16


def _logits_kernel(x_ref, wr_ref, lt_ref):
    lt_ref[...] = jax.lax.dot_general(
        wr_ref[...], x_ref[...], (((1,), (1,)), ((), ())),
        preferred_element_type=jnp.float32)


def _sc_router(lt_hbm, i1_out, i2_out, w1_out, w2_out, eid_out, na_out,
               lt_v, i1_v, i2_v, w1_v, w2_v, eid_v, misc_v):
    cid = lax.axis_index("c")
    sid = lax.axis_index("s")

    @pl.when((cid == 0) & (sid == 0))
    def _():
        pltpu.sync_copy(lt_hbm, lt_v)
        avecs = []
        for g in range(4):
            m1 = jnp.full((L,), -jnp.inf, jnp.float32)
            m2 = jnp.full((L,), -jnp.inf, jnp.float32)
            a1 = jnp.zeros((L,), jnp.int32)
            a2 = jnp.zeros((L,), jnp.int32)
            for e in range(NUM_EXPERTS):
                v = lt_v[pl.ds(e * 64 + g * L, L)]
                gt1 = v > m1
                gt2 = v > m2
                ev = jnp.full((L,), e, jnp.int32)
                m2n = jnp.where(gt1, m1, jnp.where(gt2, v, m2))
                a2n = jnp.where(gt1, a1, jnp.where(gt2, ev, a2))
                m1 = jnp.where(gt1, v, m1)
                a1 = jnp.where(gt1, ev, a1)
                m2, a2 = m2n, a2n
            ed = jnp.exp(m2 - m1)
            r = 1.0 + ed
            w1_v[pl.ds(g * L, L)] = 1.0 / r
            w2_v[pl.ds(g * L, L)] = ed / r
            i1_v[pl.ds(g * L, L)] = a1
            i2_v[pl.ds(g * L, L)] = a2
            avecs.append(a1)
            avecs.append(a2)
        zi = jnp.zeros((L,), jnp.int32)
        ids = [lax.iota(jnp.int32, L) + c * L for c in range(4)]
        actv = [jnp.zeros((L,), jnp.int32) for _ in range(4)]
        for av in avecs:
            for l in range(L):
                et = av[l]
                etv = zi + et
                for c in range(4):
                    actv[c] = jnp.maximum(
                        actv[c], jnp.where(ids[c] == etv, 1, 0))
        na_v = jnp.zeros((L,), jnp.int32)
        lastv = jnp.zeros((L,), jnp.int32)
        steps = ids
        eidv = [jnp.zeros((L,), jnp.int32) for _ in range(4)]
        for e in range(NUM_EXPERTS):
            asv = zi + actv[e // L][e % L]
            ev = jnp.full((L,), e, jnp.int32)
            for c in range(4):
                cond = jnp.where(steps[c] == na_v, asv, zi)
                eidv[c] = jnp.where(cond > 0, ev, eidv[c])
            lastv = jnp.where(asv > 0, ev, lastv)
            na_v = na_v + asv
        for c in range(4):
            eid_v[pl.ds(c * L, L)] = jnp.where(steps[c] >= na_v, lastv,
                                               eidv[c])
        misc_v[...] = na_v
        pltpu.sync_copy(i1_v, i1_out)
        pltpu.sync_copy(i2_v, i2_out)
        pltpu.sync_copy(w1_v, w1_out)
        pltpu.sync_copy(w2_v, w2_out)
        pltpu.sync_copy(eid_v.at[pl.ds(0, NUM_EXPERTS)], eid_out)
        pltpu.sync_copy(misc_v.at[pl.ds(0, 8)], na_out)


def _moe_kernel(eid_ref, na_ref, x_ref, i1_ref, i2_ref, w1_ref, w2_ref,
                wg_ref, wu_ref, wd_ref, out_ref):
    i = pl.program_id(0)
    na = na_ref[0]

    @pl.when(i < na)
    def _():
        e = eid_ref[i]
        x = x_ref[...]
        gate_h = jax.lax.dot_general(
            x, wg_ref[0], (((1,), (1,)), ((), ())),
            preferred_element_type=jnp.float32)
        up_h = jax.lax.dot_general(
            x, wu_ref[0], (((1,), (1,)), ((), ())),
            preferred_element_type=jnp.float32)
        h = (gate_h * jax.nn.sigmoid(gate_h)) * up_h
        o = jax.lax.dot_general(
            h, wd_ref[0], (((1,), (1,)), ((), ())),
            preferred_element_type=jnp.float32)
        gate = (jnp.where(i1_ref[...] == e, w1_ref[...], 0.0)
                + jnp.where(i2_ref[...] == e, w2_ref[...], 0.0))
        contrib = o * gate[:, None]

        @pl.when(i == 0)
        def _():
            out_ref[...] = contrib

        @pl.when(i > 0)
        def _():
            out_ref[...] += contrib


@jax.jit
def kernel(x, W_router, W_gate, W_up, W_down):
    orig_shape = x.shape
    x2 = x.reshape(-1, DIM)
    n = x2.shape[0]

    lt = pl.pallas_call(
        _logits_kernel,
        out_shape=jax.ShapeDtypeStruct((NUM_EXPERTS, n), jnp.float32),
    )(x2, W_router).reshape(NUM_EXPERTS * n)

    sc_router = pl.kernel(
        _sc_router,
        out_type=(
            jax.ShapeDtypeStruct((n,), jnp.int32),
            jax.ShapeDtypeStruct((n,), jnp.int32),
            jax.ShapeDtypeStruct((n,), jnp.float32),
            jax.ShapeDtypeStruct((n,), jnp.float32),
            jax.ShapeDtypeStruct((NUM_EXPERTS,), jnp.int32),
            jax.ShapeDtypeStruct((8,), jnp.int32),
        ),
        mesh=plsc.VectorSubcoreMesh(core_axis_name="c", subcore_axis_name="s"),
        scratch_types=(
            pltpu.VMEM((NUM_EXPERTS * 64,), jnp.float32),
            pltpu.VMEM((64,), jnp.int32),
            pltpu.VMEM((64,), jnp.int32),
            pltpu.VMEM((64,), jnp.float32),
            pltpu.VMEM((64,), jnp.float32),
            pltpu.VMEM((NUM_EXPERTS + 8,), jnp.int32),
            pltpu.VMEM((L,), jnp.int32),
        ),
    )
    i1, i2, w1, w2, eid, na = sc_router(lt)

    grid_spec = pltpu.PrefetchScalarGridSpec(
        num_scalar_prefetch=2,
        grid=(NUM_EXPERTS,),
        in_specs=[
            pl.BlockSpec((n, DIM), lambda i, eid, na: (0, 0)),
            pl.BlockSpec((n,), lambda i, eid, na: (0,)),
            pl.BlockSpec((n,), lambda i, eid, na: (0,)),
            pl.BlockSpec((n,), lambda i, eid, na: (0,)),
            pl.BlockSpec((n,), lambda i, eid, na: (0,)),
            pl.BlockSpec((1, HIDDEN, DIM), lambda i, eid, na: (eid[i], 0, 0)),
            pl.BlockSpec((1, HIDDEN, DIM), lambda i, eid, na: (eid[i], 0, 0)),
            pl.BlockSpec((1, DIM, HIDDEN), lambda i, eid, na: (eid[i], 0, 0)),
        ],
        out_specs=pl.BlockSpec((n, DIM), lambda i, eid, na: (0, 0)),
    )
    out = pl.pallas_call(
        _moe_kernel,
        grid_spec=grid_spec,
        out_shape=jax.ShapeDtypeStruct((n, DIM), jnp.float32),
        compiler_params=pltpu.CompilerParams(
            dimension_semantics=("arbitrary",),
        ),
    )(eid, na, x2, i1, i2, w1, w2, W_gate, W_up, W_down)

    return out.reshape(orig_shape)
